# baseline (speedup 1.0000x reference)
import functools

import jax
import jax.numpy as jnp
from jax import lax
from jax.experimental import pallas as pl
from jax.experimental.pallas import tpu as pltpu
from jax.experimental.pallas import tpu_sc as plsc

N_LABELS = 64
_LANES = 128
_ROWS = 4000
_NBLK = 125

_B = 4096
_S = 200
_NW = 32
_BW = _B // _NW


def _part_body(emits_ref, m_out, s_out):
    blk = emits_ref[...]
    bm = jnp.max(blk, axis=0, keepdims=True)
    bs = jnp.sum(jnp.exp(blk - bm), axis=0, keepdims=True)
    m_out[...] = bm.reshape(1, 1, _LANES)
    s_out[...] = bs.reshape(1, 1, _LANES)


def _tc_partials(emits2):
    return pl.pallas_call(
        _part_body,
        grid=(_NBLK,),
        in_specs=[pl.BlockSpec((_ROWS, _LANES), lambda i: (i, 0))],
        out_specs=[
            pl.BlockSpec((1, 1, _LANES), lambda i: (i, 0, 0)),
            pl.BlockSpec((1, 1, _LANES), lambda i: (i, 0, 0)),
        ],
        out_shape=[
            jax.ShapeDtypeStruct((_NBLK, 1, _LANES), jnp.float32),
            jax.ShapeDtypeStruct((_NBLK, 1, _LANES), jnp.float32),
        ],
    )(emits2)


def _comb_body(m_ref, s_ref, trans_ref, start_ref, end_ref,
               norm_ref, trans_out, start_out, end_out):
    m = m_ref[...]
    s = s_ref[...]
    mm = jnp.max(m, axis=0)
    sj = jnp.sum(s * jnp.exp(m - mm), axis=0)
    m0, m1 = mm[:, :N_LABELS], mm[:, N_LABELS:]
    s0, s1 = sj[:, :N_LABELS], sj[:, N_LABELS:]
    mc = jnp.maximum(m0, m1)
    sc = s0 * jnp.exp(m0 - mc) + s1 * jnp.exp(m1 - mc)
    norm_ref[...] = mc + jnp.log(sc)

    t = trans_ref[...]
    te = t - jnp.max(t, axis=1, keepdims=True)
    trans_out[...] = te - jnp.log(jnp.sum(jnp.exp(te), axis=1, keepdims=True))

    sv = start_ref[...]
    sve = sv - jnp.max(sv, axis=1, keepdims=True)
    start_out[...] = sve - jnp.log(jnp.sum(jnp.exp(sve), axis=1, keepdims=True))

    ev = end_ref[...]
    eve = ev - jnp.max(ev, axis=1, keepdims=True)
    end_out[...] = eve - jnp.log(jnp.sum(jnp.exp(eve), axis=1, keepdims=True))


def _tc_combine(m_part, s_part, trans, start2, end2):
    return pl.pallas_call(
        _comb_body,
        out_shape=[
            jax.ShapeDtypeStruct((1, N_LABELS), jnp.float32),
            jax.ShapeDtypeStruct((N_LABELS, N_LABELS), jnp.float32),
            jax.ShapeDtypeStruct((1, N_LABELS), jnp.float32),
            jax.ShapeDtypeStruct((1, N_LABELS), jnp.float32),
        ],
    )(m_part, s_part, trans, start2, end2)


def _sc_gather_fn(table_hbm, wt_hbm, norm_hbm, out_hbm,
                  idx_v, rows0, rows1, ob0, ob1, norm_v, sem0, sem1):
    wid = lax.axis_index("s") * 2 + lax.axis_index("c")
    b0 = wid * _BW

    pltpu.sync_copy(norm_hbm, norm_v)
    nvec = [norm_v[pl.ds(16 * k, 16)] for k in range(4)]
    pltpu.sync_copy(wt_hbm.at[:, pl.ds(b0, _BW)], idx_v)
    iota16 = lax.iota(jnp.int32, 16)

    def g_start(s, rows_v, sem):
        pltpu.make_async_copy(
            table_hbm.at[idx_v.at[s]], rows_v, sem).start()

    def g_wait(s, rows_v, sem):
        pltpu.make_async_copy(
            table_hbm.at[idx_v.at[s]], rows_v, sem).wait()

    def tsub(rows_v, ob):
        def body(b, c):
            bcol = lax.broadcast(b, (16,)).astype(jnp.int32)
            for jg in range(4):
                vals = rows_v[b, pl.ds(16 * jg, 16)] - nvec[jg]
                plsc.store_scatter(ob, [iota16 + 16 * jg, bcol], vals)
            return c
        lax.fori_loop(0, _BW, body, 0)

    g_start(0, rows0, sem0)

    def pair(i, c):
        s0 = 2 * i
        s1 = s0 + 1

        g_start(s1, rows1, sem1)

        g_wait(s0, rows0, sem0)
        tsub(rows0, ob0)
        pltpu.sync_copy(ob0, out_hbm.at[s0, :, pl.ds(b0, _BW)])

        @pl.when(i < _S // 2 - 1)
        def _():
            g_start(s0 + 2, rows0, sem0)

        g_wait(s1, rows1, sem1)
        tsub(rows1, ob1)
        pltpu.sync_copy(ob1, out_hbm.at[s1, :, pl.ds(b0, _BW)])
        return c

    lax.fori_loop(0, _S // 2, pair, 0)


@functools.lru_cache(maxsize=1)
def _sc_gather():
    return pl.kernel(
        _sc_gather_fn,
        out_type=jax.ShapeDtypeStruct((_S, N_LABELS, _B), jnp.float32),
        mesh=plsc.VectorSubcoreMesh(core_axis_name="c", subcore_axis_name="s"),
        compiler_params=pltpu.CompilerParams(
            use_tc_tiling_on_sc=False, needs_layout_passes=False),
        scratch_types=[
            pltpu.VMEM((_S, _BW), jnp.int32),
            pltpu.VMEM((_BW, N_LABELS), jnp.float32),
            pltpu.VMEM((_BW, N_LABELS), jnp.float32),
            pltpu.VMEM((N_LABELS, _BW), jnp.float32),
            pltpu.VMEM((N_LABELS, _BW), jnp.float32),
            pltpu.VMEM((N_LABELS,), jnp.float32),
            pltpu.SemaphoreType.DMA,
            pltpu.SemaphoreType.DMA,
        ],
    )


def kernel(words, mask, emits, transitions, start, end):
    del mask
    words_t = words.astype(jnp.int32).T
    emits2 = emits.reshape(-1, _LANES)
    m_part, s_part = _tc_partials(emits2)
    norm, trans_ls, start_ls, end_ls = _tc_combine(
        m_part, s_part, transitions, start.reshape(1, -1), end.reshape(1, -1))
    out3 = _sc_gather()(emits, words_t, norm.reshape(N_LABELS))
    scores = jnp.transpose(out3, (2, 0, 1))
    return scores, trans_ls, start_ls.reshape(-1), end_ls.reshape(-1)


# speedup vs baseline: 1.1956x; 1.1956x over previous
import functools

import jax
import jax.numpy as jnp
from jax import lax
from jax.experimental import pallas as pl
from jax.experimental.pallas import tpu as pltpu
from jax.experimental.pallas import tpu_sc as plsc

N_LABELS = 64
_LANES = 128
_ROWS = 4000
_NBLK = 125

_B = 4096
_S = 200
_NW = 32
_BW = _B // _NW


def _part_body(emits_ref, m_out, s_out):
    blk = emits_ref[...]
    bm = jnp.max(blk, axis=0, keepdims=True)
    bs = jnp.sum(jnp.exp(blk - bm), axis=0, keepdims=True)
    m_out[...] = bm.reshape(1, 1, _LANES)
    s_out[...] = bs.reshape(1, 1, _LANES)


def _tc_partials(emits2):
    return pl.pallas_call(
        _part_body,
        grid=(_NBLK,),
        in_specs=[pl.BlockSpec((_ROWS, _LANES), lambda i: (i, 0))],
        out_specs=[
            pl.BlockSpec((1, 1, _LANES), lambda i: (i, 0, 0)),
            pl.BlockSpec((1, 1, _LANES), lambda i: (i, 0, 0)),
        ],
        out_shape=[
            jax.ShapeDtypeStruct((_NBLK, 1, _LANES), jnp.float32),
            jax.ShapeDtypeStruct((_NBLK, 1, _LANES), jnp.float32),
        ],
    )(emits2)


def _comb_body(m_ref, s_ref, trans_ref, start_ref, end_ref,
               norm_ref, trans_out, start_out, end_out):
    m = m_ref[...]
    s = s_ref[...]
    mm = jnp.max(m, axis=0)
    sj = jnp.sum(s * jnp.exp(m - mm), axis=0)
    m0, m1 = mm[:, :N_LABELS], mm[:, N_LABELS:]
    s0, s1 = sj[:, :N_LABELS], sj[:, N_LABELS:]
    mc = jnp.maximum(m0, m1)
    sc = s0 * jnp.exp(m0 - mc) + s1 * jnp.exp(m1 - mc)
    norm_ref[...] = mc + jnp.log(sc)

    t = trans_ref[...]
    te = t - jnp.max(t, axis=1, keepdims=True)
    trans_out[...] = te - jnp.log(jnp.sum(jnp.exp(te), axis=1, keepdims=True))

    sv = start_ref[...]
    sve = sv - jnp.max(sv, axis=1, keepdims=True)
    start_out[...] = sve - jnp.log(jnp.sum(jnp.exp(sve), axis=1, keepdims=True))

    ev = end_ref[...]
    eve = ev - jnp.max(ev, axis=1, keepdims=True)
    end_out[...] = eve - jnp.log(jnp.sum(jnp.exp(eve), axis=1, keepdims=True))


def _tc_combine(m_part, s_part, trans, start2, end2):
    return pl.pallas_call(
        _comb_body,
        out_shape=[
            jax.ShapeDtypeStruct((1, N_LABELS), jnp.float32),
            jax.ShapeDtypeStruct((N_LABELS, N_LABELS), jnp.float32),
            jax.ShapeDtypeStruct((1, N_LABELS), jnp.float32),
            jax.ShapeDtypeStruct((1, N_LABELS), jnp.float32),
        ],
    )(m_part, s_part, trans, start2, end2)


def _sc_gather_fn(table_hbm, wt_hbm, norm_hbm, out_hbm,
                  idx_v, rows0, rows1, ob0, ob1, norm_v, sem0, sem1):
    wid = lax.axis_index("s") * 2 + lax.axis_index("c")
    b0 = wid * _BW

    pltpu.sync_copy(norm_hbm, norm_v)
    nvec = [norm_v[pl.ds(16 * k, 16)] for k in range(4)]
    pltpu.sync_copy(wt_hbm.at[:, pl.ds(b0, _BW)], idx_v)
    iota16 = lax.iota(jnp.int32, 16)
    base_idx = [(iota16 + 16 * jg) * 128 for jg in range(4)]

    def g_start(s, rows_v, sem):
        pltpu.make_async_copy(
            table_hbm.at[idx_v.at[s]], rows_v, sem).start()

    def g_wait(s, rows_v, sem):
        pltpu.make_async_copy(
            table_hbm.at[idx_v.at[s]], rows_v, sem).wait()

    def tsub(rows_v, ob):
        @plsc.parallel_loop(0, _BW, 1, unroll=8,
                            carry=jnp.zeros((16,), jnp.int32))
        def _(b, bcol):
            for jg in range(4):
                vals = rows_v[b, pl.ds(16 * jg, 16)] - nvec[jg]
                plsc.store_scatter(ob, [iota16 + 16 * jg, bcol], vals)
            return bcol + 1

    g_start(0, rows0, sem0)

    def pair(i, c):
        s0 = 2 * i
        s1 = s0 + 1

        g_start(s1, rows1, sem1)

        g_wait(s0, rows0, sem0)
        tsub(rows0, ob0)
        pltpu.sync_copy(ob0, out_hbm.at[s0, :, pl.ds(b0, _BW)])

        @pl.when(i < _S // 2 - 1)
        def _():
            g_start(s0 + 2, rows0, sem0)

        g_wait(s1, rows1, sem1)
        tsub(rows1, ob1)
        pltpu.sync_copy(ob1, out_hbm.at[s1, :, pl.ds(b0, _BW)])
        return c

    lax.fori_loop(0, _S // 2, pair, 0)


@functools.lru_cache(maxsize=1)
def _sc_gather():
    return pl.kernel(
        _sc_gather_fn,
        out_type=jax.ShapeDtypeStruct((_S, N_LABELS, _B), jnp.float32),
        mesh=plsc.VectorSubcoreMesh(core_axis_name="c", subcore_axis_name="s"),
        compiler_params=pltpu.CompilerParams(
            use_tc_tiling_on_sc=False, needs_layout_passes=False),
        scratch_types=[
            pltpu.VMEM((_S, _BW), jnp.int32),
            pltpu.VMEM((_BW, N_LABELS), jnp.float32),
            pltpu.VMEM((_BW, N_LABELS), jnp.float32),
            pltpu.VMEM((N_LABELS, _BW), jnp.float32),
            pltpu.VMEM((N_LABELS, _BW), jnp.float32),
            pltpu.VMEM((N_LABELS,), jnp.float32),
            pltpu.SemaphoreType.DMA,
            pltpu.SemaphoreType.DMA,
        ],
    )


def kernel(words, mask, emits, transitions, start, end):
    del mask
    words_t = words.astype(jnp.int32).T
    emits2 = emits.reshape(-1, _LANES)
    m_part, s_part = _tc_partials(emits2)
    norm, trans_ls, start_ls, end_ls = _tc_combine(
        m_part, s_part, transitions, start.reshape(1, -1), end.reshape(1, -1))
    out3 = _sc_gather()(emits, words_t, norm.reshape(N_LABELS))
    scores = jnp.transpose(out3, (2, 0, 1))
    return scores, trans_ls, start_ls.reshape(-1), end_ls.reshape(-1)


# speedup vs baseline: 1.6734x; 1.3996x over previous
import functools

import jax
import jax.numpy as jnp
from jax import lax
from jax.experimental import pallas as pl
from jax.experimental.pallas import tpu as pltpu
from jax.experimental.pallas import tpu_sc as plsc

N_LABELS = 64
_LANES = 128
_ROWS = 25000
_NBLK = 20

_B = 4096
_S = 200
_NW = 32
_BW = _B // _NW
_OBP = 129


def _part_body(emits_ref, m_out, s_out):
    blk = emits_ref[...]
    bm = jnp.max(blk, axis=0, keepdims=True)
    bs = jnp.sum(jnp.exp(blk - bm), axis=0, keepdims=True)
    m_out[...] = bm.reshape(1, 1, _LANES)
    s_out[...] = bs.reshape(1, 1, _LANES)


def _tc_partials(emits2):
    return pl.pallas_call(
        _part_body,
        grid=(_NBLK,),
        in_specs=[pl.BlockSpec((_ROWS, _LANES), lambda i: (i, 0))],
        out_specs=[
            pl.BlockSpec((1, 1, _LANES), lambda i: (i, 0, 0)),
            pl.BlockSpec((1, 1, _LANES), lambda i: (i, 0, 0)),
        ],
        out_shape=[
            jax.ShapeDtypeStruct((_NBLK, 1, _LANES), jnp.float32),
            jax.ShapeDtypeStruct((_NBLK, 1, _LANES), jnp.float32),
        ],
    )(emits2)


def _comb_body(m_ref, s_ref, trans_ref, start_ref, end_ref,
               norm_ref, trans_out, start_out, end_out):
    m = m_ref[...]
    s = s_ref[...]
    mm = jnp.max(m, axis=0)
    sj = jnp.sum(s * jnp.exp(m - mm), axis=0)
    m0, m1 = mm[:, :N_LABELS], mm[:, N_LABELS:]
    s0, s1 = sj[:, :N_LABELS], sj[:, N_LABELS:]
    mc = jnp.maximum(m0, m1)
    sc = s0 * jnp.exp(m0 - mc) + s1 * jnp.exp(m1 - mc)
    norm_ref[...] = mc + jnp.log(sc)

    t = trans_ref[...]
    te = t - jnp.max(t, axis=1, keepdims=True)
    trans_out[...] = te - jnp.log(jnp.sum(jnp.exp(te), axis=1, keepdims=True))

    sv = start_ref[...]
    sve = sv - jnp.max(sv, axis=1, keepdims=True)
    start_out[...] = sve - jnp.log(jnp.sum(jnp.exp(sve), axis=1, keepdims=True))

    ev = end_ref[...]
    eve = ev - jnp.max(ev, axis=1, keepdims=True)
    end_out[...] = eve - jnp.log(jnp.sum(jnp.exp(eve), axis=1, keepdims=True))


def _tc_combine(m_part, s_part, trans, start2, end2):
    return pl.pallas_call(
        _comb_body,
        out_shape=[
            jax.ShapeDtypeStruct((1, N_LABELS), jnp.float32),
            jax.ShapeDtypeStruct((N_LABELS, N_LABELS), jnp.float32),
            jax.ShapeDtypeStruct((1, N_LABELS), jnp.float32),
            jax.ShapeDtypeStruct((1, N_LABELS), jnp.float32),
        ],
    )(m_part, s_part, trans, start2, end2)


def _sc_gather_fn(table_hbm, wt_hbm, norm_hbm, out_hbm,
                  idx_v, rows0, rows1, ob0, ob1, norm_v, sem0, sem1):
    wid = lax.axis_index("s") * 2 + lax.axis_index("c")
    b0 = wid * _BW

    pltpu.sync_copy(norm_hbm, norm_v)
    nvec = [norm_v[pl.ds(16 * k, 16)] for k in range(4)]
    pltpu.sync_copy(wt_hbm.at[:, pl.ds(b0, _BW)], idx_v)
    iota16 = lax.iota(jnp.int32, 16)

    def g_start(s, rows_v, sem):
        pltpu.make_async_copy(
            table_hbm.at[idx_v.at[s]], rows_v, sem).start()

    def g_wait(s, rows_v, sem):
        pltpu.make_async_copy(
            table_hbm.at[idx_v.at[s]], rows_v, sem).wait()

    def tsub(rows_v, ob):
        @plsc.parallel_loop(0, _BW, 1, unroll=8,
                            carry=jnp.zeros((16,), jnp.int32))
        def _(b, bcol):
            for jg in range(4):
                vals = rows_v[b, pl.ds(16 * jg, 16)] - nvec[jg]
                plsc.store_scatter(ob, [iota16 + 16 * jg, bcol], vals)
            return bcol + 1

    g_start(0, rows0, sem0)

    def pair(i, c):
        s0 = 2 * i
        s1 = s0 + 1

        g_start(s1, rows1, sem1)

        g_wait(s0, rows0, sem0)
        tsub(rows0, ob0)
        pltpu.sync_copy(ob0.at[:, pl.ds(0, _BW)],
                        out_hbm.at[s0, :, pl.ds(b0, _BW)])

        @pl.when(i < _S // 2 - 1)
        def _():
            g_start(s0 + 2, rows0, sem0)

        g_wait(s1, rows1, sem1)
        tsub(rows1, ob1)
        pltpu.sync_copy(ob1.at[:, pl.ds(0, _BW)],
                        out_hbm.at[s1, :, pl.ds(b0, _BW)])
        return c

    lax.fori_loop(0, _S // 2, pair, 0)


@functools.lru_cache(maxsize=1)
def _sc_gather():
    return pl.kernel(
        _sc_gather_fn,
        out_type=jax.ShapeDtypeStruct((_S, N_LABELS, _B), jnp.float32),
        mesh=plsc.VectorSubcoreMesh(core_axis_name="c", subcore_axis_name="s"),
        compiler_params=pltpu.CompilerParams(
            use_tc_tiling_on_sc=False, needs_layout_passes=False),
        scratch_types=[
            pltpu.VMEM((_S, _BW), jnp.int32),
            pltpu.VMEM((_BW, N_LABELS), jnp.float32),
            pltpu.VMEM((_BW, N_LABELS), jnp.float32),
            pltpu.VMEM((N_LABELS, _OBP), jnp.float32),
            pltpu.VMEM((N_LABELS, _OBP), jnp.float32),
            pltpu.VMEM((N_LABELS,), jnp.float32),
            pltpu.SemaphoreType.DMA,
            pltpu.SemaphoreType.DMA,
        ],
    )


def kernel(words, mask, emits, transitions, start, end):
    del mask
    words_t = words.astype(jnp.int32).T
    emits2 = emits.reshape(-1, _LANES)
    m_part, s_part = _tc_partials(emits2)
    norm, trans_ls, start_ls, end_ls = _tc_combine(
        m_part, s_part, transitions, start.reshape(1, -1), end.reshape(1, -1))
    out3 = _sc_gather()(emits, words_t, norm.reshape(N_LABELS))
    scores = jnp.transpose(out3, (2, 0, 1))
    return scores, trans_ls, start_ls.reshape(-1), end_ls.reshape(-1)


# speedup vs baseline: 1.9455x; 1.1626x over previous
import functools

import jax
import jax.numpy as jnp
from jax import lax
from jax.experimental import pallas as pl
from jax.experimental.pallas import tpu as pltpu
from jax.experimental.pallas import tpu_sc as plsc

N_LABELS = 64
_ROWS = 50000
_NBLK = 20

_B = 4096
_S = 200
_NW = 32
_BW = _B // _NW
_OBP = 129


def _part_body(emits_ref, m_out, s_out):
    blk = emits_ref[...]
    bm = jnp.max(blk, axis=0, keepdims=True)
    bs = jnp.sum(jnp.exp(blk - bm), axis=0, keepdims=True)
    m_out[...] = bm.reshape(1, 1, N_LABELS)
    s_out[...] = bs.reshape(1, 1, N_LABELS)


def _tc_partials(emits):
    return pl.pallas_call(
        _part_body,
        grid=(_NBLK,),
        in_specs=[pl.BlockSpec((_ROWS, N_LABELS), lambda i: (i, 0))],
        out_specs=[
            pl.BlockSpec((1, 1, N_LABELS), lambda i: (i, 0, 0)),
            pl.BlockSpec((1, 1, N_LABELS), lambda i: (i, 0, 0)),
        ],
        out_shape=[
            jax.ShapeDtypeStruct((_NBLK, 1, N_LABELS), jnp.float32),
            jax.ShapeDtypeStruct((_NBLK, 1, N_LABELS), jnp.float32),
        ],
    )(emits)


def _comb_body(m_ref, s_ref, trans_ref, start_ref, end_ref,
               norm_ref, trans_out, start_out, end_out):
    m = m_ref[...]
    s = s_ref[...]
    mc = jnp.max(m, axis=0)
    sc = jnp.sum(s * jnp.exp(m - mc), axis=0)
    norm_ref[...] = mc + jnp.log(sc)

    t = trans_ref[...]
    te = t - jnp.max(t, axis=1, keepdims=True)
    trans_out[...] = te - jnp.log(jnp.sum(jnp.exp(te), axis=1, keepdims=True))

    sv = start_ref[...]
    sve = sv - jnp.max(sv, axis=1, keepdims=True)
    start_out[...] = sve - jnp.log(jnp.sum(jnp.exp(sve), axis=1, keepdims=True))

    ev = end_ref[...]
    eve = ev - jnp.max(ev, axis=1, keepdims=True)
    end_out[...] = eve - jnp.log(jnp.sum(jnp.exp(eve), axis=1, keepdims=True))


def _tc_combine(m_part, s_part, trans, start2, end2):
    return pl.pallas_call(
        _comb_body,
        out_shape=[
            jax.ShapeDtypeStruct((1, N_LABELS), jnp.float32),
            jax.ShapeDtypeStruct((N_LABELS, N_LABELS), jnp.float32),
            jax.ShapeDtypeStruct((1, N_LABELS), jnp.float32),
            jax.ShapeDtypeStruct((1, N_LABELS), jnp.float32),
        ],
    )(m_part, s_part, trans, start2, end2)


def _sc_gather_fn(table_hbm, wt_hbm, norm_hbm, out_hbm,
                  idx_v, rows0, rows1, ob0, ob1, norm_v, sem0, sem1):
    wid = lax.axis_index("s") * 2 + lax.axis_index("c")
    b0 = wid * _BW

    pltpu.sync_copy(norm_hbm, norm_v)
    nvec = [norm_v[pl.ds(16 * k, 16)] for k in range(4)]
    pltpu.sync_copy(wt_hbm.at[:, pl.ds(b0, _BW)], idx_v)
    iota16 = lax.iota(jnp.int32, 16)

    def g_start(s, rows_v, sem):
        pltpu.make_async_copy(
            table_hbm.at[idx_v.at[s]], rows_v, sem).start()

    def g_wait(s, rows_v, sem):
        pltpu.make_async_copy(
            table_hbm.at[idx_v.at[s]], rows_v, sem).wait()

    def tsub(rows_v, ob):
        @plsc.parallel_loop(0, _BW, 1, unroll=8,
                            carry=jnp.zeros((16,), jnp.int32))
        def _(b, bcol):
            for jg in range(4):
                vals = rows_v[b, pl.ds(16 * jg, 16)] - nvec[jg]
                plsc.store_scatter(ob, [iota16 + 16 * jg, bcol], vals)
            return bcol + 1

    g_start(0, rows0, sem0)

    def pair(i, c):
        s0 = 2 * i
        s1 = s0 + 1

        g_start(s1, rows1, sem1)

        g_wait(s0, rows0, sem0)
        tsub(rows0, ob0)
        pltpu.sync_copy(ob0.at[:, pl.ds(0, _BW)],
                        out_hbm.at[s0, :, pl.ds(b0, _BW)])

        @pl.when(i < _S // 2 - 1)
        def _():
            g_start(s0 + 2, rows0, sem0)

        g_wait(s1, rows1, sem1)
        tsub(rows1, ob1)
        pltpu.sync_copy(ob1.at[:, pl.ds(0, _BW)],
                        out_hbm.at[s1, :, pl.ds(b0, _BW)])
        return c

    lax.fori_loop(0, _S // 2, pair, 0)


@functools.lru_cache(maxsize=1)
def _sc_gather():
    return pl.kernel(
        _sc_gather_fn,
        out_type=jax.ShapeDtypeStruct((_S, N_LABELS, _B), jnp.float32),
        mesh=plsc.VectorSubcoreMesh(core_axis_name="c", subcore_axis_name="s"),
        compiler_params=pltpu.CompilerParams(
            use_tc_tiling_on_sc=False, needs_layout_passes=False),
        scratch_types=[
            pltpu.VMEM((_S, _BW), jnp.int32),
            pltpu.VMEM((_BW, N_LABELS), jnp.float32),
            pltpu.VMEM((_BW, N_LABELS), jnp.float32),
            pltpu.VMEM((N_LABELS, _OBP), jnp.float32),
            pltpu.VMEM((N_LABELS, _OBP), jnp.float32),
            pltpu.VMEM((N_LABELS,), jnp.float32),
            pltpu.SemaphoreType.DMA,
            pltpu.SemaphoreType.DMA,
        ],
    )


def kernel(words, mask, emits, transitions, start, end):
    del mask
    words_t = words.astype(jnp.int32).T
    m_part, s_part = _tc_partials(emits)
    norm, trans_ls, start_ls, end_ls = _tc_combine(
        m_part, s_part, transitions, start.reshape(1, -1), end.reshape(1, -1))
    out3 = _sc_gather()(emits, words_t, norm.reshape(N_LABELS))
    scores = jnp.transpose(out3, (2, 0, 1))
    return scores, trans_ls, start_ls.reshape(-1), end_ls.reshape(-1)
